# baseline (speedup 1.0000x reference)
import functools

import jax
import jax.numpy as jnp
from jax import lax
from jax.experimental import pallas as pl
from jax.experimental.pallas import tpu as pltpu
from jax.experimental.pallas import tpu_sc as plsc

B, CDD, HIS, T, H, K = 64, 5, 50, 20, 256, 5
VOCAB = 30000
THRESHOLD = 0.1

CDD_P = 8
HIS_P = 56
SLOTS = CDD_P + HIS_P
NROWS = B * SLOTS
NW = 32
ROWS_PER_W = NROWS // NW
SEGS_PER_CHUNK = 4
TOK_PER_CHUNK = SEGS_PER_CHUNK * T
CHUNKS_PER_W = ROWS_PER_W // SEGS_PER_CHUNK
TOK_PER_W = ROWS_PER_W * T


def _proj_body(x_ref, w_ref, b_ref, o_ref):
    h = jnp.dot(x_ref[...], w_ref[...], preferred_element_type=jnp.float32)
    o_ref[...] = jnp.tanh(h + b_ref[...]) * (1.0 / T)


def _project_table(emb_table, w_enc, b_enc):
    rows = 1200
    return pl.pallas_call(
        _proj_body,
        grid=(VOCAB // rows,),
        in_specs=[
            pl.BlockSpec((rows, H), lambda i: (i, 0)),
            pl.BlockSpec((H, H), lambda i: (0, 0)),
            pl.BlockSpec((1, H), lambda i: (0, 0)),
        ],
        out_specs=pl.BlockSpec((rows, H), lambda i: (i, 0)),
        out_shape=jax.ShapeDtypeStruct((VOCAB, H), jnp.float32),
    )(emb_table, w_enc, b_enc.reshape(1, H))


def _sc_body(tok_hbm, p_hbm, out_hbm, idx_v, buf_a, buf_b, stage_v,
             sem_a, sem_b):
    wid = lax.axis_index("s") * 2 + lax.axis_index("c")
    pltpu.sync_copy(tok_hbm.at[pl.ds(wid * TOK_PER_W, TOK_PER_W)], idx_v)

    def fire(chunk, buf, sem):
        pltpu.async_copy(
            p_hbm.at[idx_v.at[pl.ds(chunk * TOK_PER_CHUNK, TOK_PER_CHUNK)]],
            buf, sem)

    def drain(buf, sem):
        pltpu.make_async_copy(p_hbm.at[pl.ds(0, TOK_PER_CHUNK)], buf,
                              sem).wait()

    def process(buf, chunk):
        for k in range(SEGS_PER_CHUNK):
            accs = [buf[k * T, c * 16:(c + 1) * 16] for c in range(16)]
            for t in range(1, T):
                accs = [accs[c] + buf[k * T + t, c * 16:(c + 1) * 16]
                        for c in range(16)]
            slot = chunk * SEGS_PER_CHUNK + k
            for c in range(16):
                stage_v[slot, c * 16:(c + 1) * 16] = accs[c]

    fire(0, buf_a, sem_a)
    fire(1, buf_b, sem_b)

    def pair(i, _):
        drain(buf_a, sem_a)
        process(buf_a, 2 * i)

        @pl.when(2 * i + 2 < CHUNKS_PER_W)
        def _():
            fire(2 * i + 2, buf_a, sem_a)

        drain(buf_b, sem_b)
        process(buf_b, 2 * i + 1)

        @pl.when(2 * i + 3 < CHUNKS_PER_W)
        def _():
            fire(2 * i + 3, buf_b, sem_b)
        return 0

    lax.fori_loop(0, CHUNKS_PER_W // 2, pair, 0)
    pltpu.sync_copy(stage_v, out_hbm.at[pl.ds(wid * ROWS_PER_W, ROWS_PER_W)])


@functools.partial(
    pl.kernel,
    mesh=plsc.VectorSubcoreMesh(core_axis_name="c", subcore_axis_name="s"),
    out_type=jax.ShapeDtypeStruct((NROWS, H), jnp.float32),
    scratch_types=[
        pltpu.VMEM((TOK_PER_W,), jnp.int32),
        pltpu.VMEM((TOK_PER_CHUNK, H), jnp.float32),
        pltpu.VMEM((TOK_PER_CHUNK, H), jnp.float32),
        pltpu.VMEM((ROWS_PER_W, H), jnp.float32),
        pltpu.SemaphoreType.DMA,
        pltpu.SemaphoreType.DMA,
    ],
)
def _gather_mean(tok_hbm, p_hbm, out_hbm, idx_v, buf_a, buf_b, stage_v,
                 sem_a, sem_b):
    _sc_body(tok_hbm, p_hbm, out_hbm, idx_v, buf_a, buf_b, stage_v,
             sem_a, sem_b)


def _head_body(rep_ref, wsel_ref, bsel_ref, wint_ref, bint_ref,
               w1_ref, b1_ref, w2t_ref, o_ref):
    rep_b = rep_ref[...]
    sel = jnp.dot(rep_b, wsel_ref[...],
                  preferred_element_type=jnp.float32) + bsel_ref[...]
    norm = jnp.sqrt(jnp.sum(sel * sel, axis=1, keepdims=True))
    seln = sel / jnp.maximum(norm, 1e-12)
    cddp = seln[0:CDD_P]
    hisp = seln[CDD_P:SLOTS]
    attn = lax.dot_general(cddp, hisp, (((1,), (1,)), ((), ())),
                           preferred_element_type=jnp.float32)

    col = lax.broadcasted_iota(jnp.int32, (CDD_P, HIS_P), 1)
    a = jnp.where(col < HIS, attn, -1e30)
    w = jnp.zeros((CDD_P, HIS_P), jnp.float32)
    for _ in range(K):
        m = jnp.max(a, axis=1, keepdims=True)
        eq = a == m
        first = jnp.min(jnp.where(eq, col, HIS_P), axis=1, keepdims=True)
        onehot = col == first
        w = w + jnp.where(onehot & (m >= THRESHOLD), m, 0.0)
        a = jnp.where(onehot, -1e30, a)

    rvalid = lax.broadcasted_iota(jnp.int32, (HIS_P, H), 0) < HIS
    his_real = jnp.where(rvalid, rep_b[CDD_P:SLOTS], 0.0)
    hisv = jnp.dot(w, his_real,
                   preferred_element_type=jnp.float32) * (1.0 / K)
    fus = jnp.maximum(
        jnp.dot(rep_b[0:CDD_P] * hisv, wint_ref[...],
                preferred_element_type=jnp.float32) + bint_ref[...], 0.0)
    h1 = jnp.maximum(
        jnp.dot(fus, w1_ref[...],
                preferred_element_type=jnp.float32) + b1_ref[...], 0.0)
    score = lax.dot_general(w2t_ref[...], h1, (((1,), (1,)), ((), ())),
                            preferred_element_type=jnp.float32)

    lane = lax.broadcasted_iota(jnp.int32, (1, CDD_P), 1)
    sm = jnp.where(lane < CDD, score, -1e30)
    mx = jnp.max(sm, axis=1, keepdims=True)
    lse = jnp.log(jnp.sum(jnp.exp(sm - mx), axis=1, keepdims=True)) + mx
    o_ref[...] = (score - lse)[None]


def _head(rep, w_sel, b_sel, w_int, b_int, w1, b1, w2):
    return pl.pallas_call(
        _head_body,
        grid=(B,),
        in_specs=[
            pl.BlockSpec((SLOTS, H), lambda b: (b, 0)),
            pl.BlockSpec((H, H), lambda b: (0, 0)),
            pl.BlockSpec((1, H), lambda b: (0, 0)),
            pl.BlockSpec((H, H), lambda b: (0, 0)),
            pl.BlockSpec((1, H), lambda b: (0, 0)),
            pl.BlockSpec((H, H // 2), lambda b: (0, 0)),
            pl.BlockSpec((1, H // 2), lambda b: (0, 0)),
            pl.BlockSpec((1, H // 2), lambda b: (0, 0)),
        ],
        out_specs=pl.BlockSpec((1, 1, CDD_P), lambda b: (b, 0, 0)),
        out_shape=jax.ShapeDtypeStruct((B, 1, CDD_P), jnp.float32),
    )(rep, w_sel, b_sel.reshape(1, H), w_int, b_int.reshape(1, H),
      w1, b1.reshape(1, H // 2), w2.reshape(1, H // 2))


def kernel(cdd_encoded_index, his_encoded_index, emb_table, W_enc, b_enc,
           W_sel, b_sel, W_int, b_int, W1, b1, W2, b2):
    p = _project_table(emb_table, W_enc, b_enc)
    zc = jnp.zeros((B, CDD_P - CDD, T), jnp.int32)
    zh = jnp.zeros((B, HIS_P - HIS, T), jnp.int32)
    tok = jnp.concatenate(
        [cdd_encoded_index.astype(jnp.int32), zc,
         his_encoded_index.astype(jnp.int32), zh], axis=1).reshape(-1)
    rep = _gather_mean(tok, p)
    out = _head(rep, W_sel, b_sel, W_int, b_int, W1, b1, W2)
    return out.reshape(B, CDD_P)[:, :CDD]


# speedup vs baseline: 1.0500x; 1.0500x over previous
import functools

import jax
import jax.numpy as jnp
from jax import lax
from jax.experimental import pallas as pl
from jax.experimental.pallas import tpu as pltpu
from jax.experimental.pallas import tpu_sc as plsc

B, CDD, HIS, T, H, K = 64, 5, 50, 20, 256, 5
VOCAB = 30000
THRESHOLD = 0.1

CDD_P = 8
HIS_P = 56
SLOTS = CDD_P + HIS_P
NROWS = B * SLOTS
NW = 32
ROWS_PER_W = NROWS // NW
SEGS_PER_CHUNK = 16
TOK_PER_CHUNK = SEGS_PER_CHUNK * T
CHUNKS_PER_W = ROWS_PER_W // SEGS_PER_CHUNK
TOK_PER_W = ROWS_PER_W * T
HW = H // 2
QSCALE = float(1 << 14)
DECODE = 1.0 / float(1 << 30)


def _proj_body(x_ref, w_ref, b_ref, o_ref):
    h = jnp.dot(x_ref[...], w_ref[...], preferred_element_type=jnp.float32)
    v = jnp.tanh(h + b_ref[...]) * (1.0 / T)
    q = jnp.rint(v * QSCALE).astype(jnp.int32)
    lo = q[:, :HW] & jnp.int32(0xFFFF)
    hi = q[:, HW:] << 16
    o_ref[...] = lo | hi


def _project_table(emb_table, w_enc, b_enc):
    rows = 1200
    return pl.pallas_call(
        _proj_body,
        grid=(VOCAB // rows,),
        in_specs=[
            pl.BlockSpec((rows, H), lambda i: (i, 0)),
            pl.BlockSpec((H, H), lambda i: (0, 0)),
            pl.BlockSpec((1, H), lambda i: (0, 0)),
        ],
        out_specs=pl.BlockSpec((rows, HW), lambda i: (i, 0)),
        out_shape=jax.ShapeDtypeStruct((VOCAB, HW), jnp.int32),
    )(emb_table.astype(jnp.bfloat16), w_enc.astype(jnp.bfloat16),
      b_enc.reshape(1, H))


def _sc_body(tok_hbm, p_hbm, out_hbm, idx_v, buf_a, buf_b, stage_v,
             sem_a, sem_b):
    wid = lax.axis_index("s") * 2 + lax.axis_index("c")
    pltpu.sync_copy(tok_hbm.at[pl.ds(wid * TOK_PER_W, TOK_PER_W)], idx_v)

    def fire(chunk, buf, sem):
        pltpu.async_copy(
            p_hbm.at[idx_v.at[pl.ds(chunk * TOK_PER_CHUNK, TOK_PER_CHUNK)]],
            buf, sem)

    def drain(buf, sem):
        pltpu.make_async_copy(p_hbm.at[pl.ds(0, TOK_PER_CHUNK)], buf,
                              sem).wait()

    def process(buf, chunk):
        for k in range(SEGS_PER_CHUNK):
            x0 = [buf[k * T, c * 16:(c + 1) * 16] for c in range(8)]
            lo = [x << 16 for x in x0]
            hi = [x & jnp.int32(-65536) for x in x0]
            for t in range(1, T):
                xt = [buf[k * T + t, c * 16:(c + 1) * 16] for c in range(8)]
                lo = [lo[c] + (xt[c] << 16) for c in range(8)]
                hi = [hi[c] + (xt[c] & jnp.int32(-65536)) for c in range(8)]
            slot = chunk * SEGS_PER_CHUNK + k
            for c in range(8):
                stage_v[slot, c * 16:(c + 1) * 16] = lo[c]
                stage_v[slot, HW + c * 16:HW + (c + 1) * 16] = hi[c]

    fire(0, buf_a, sem_a)
    fire(1, buf_b, sem_b)

    def pair(i, _):
        drain(buf_a, sem_a)
        process(buf_a, 2 * i)

        @pl.when(2 * i + 2 < CHUNKS_PER_W)
        def _():
            fire(2 * i + 2, buf_a, sem_a)

        drain(buf_b, sem_b)
        process(buf_b, 2 * i + 1)

        @pl.when(2 * i + 3 < CHUNKS_PER_W)
        def _():
            fire(2 * i + 3, buf_b, sem_b)
        return 0

    lax.fori_loop(0, CHUNKS_PER_W // 2, pair, 0)
    pltpu.sync_copy(stage_v, out_hbm.at[pl.ds(wid * ROWS_PER_W, ROWS_PER_W)])


@functools.partial(
    pl.kernel,
    mesh=plsc.VectorSubcoreMesh(core_axis_name="c", subcore_axis_name="s"),
    out_type=jax.ShapeDtypeStruct((NROWS, H), jnp.int32),
    scratch_types=[
        pltpu.VMEM((TOK_PER_W,), jnp.int32),
        pltpu.VMEM((TOK_PER_CHUNK, HW), jnp.int32),
        pltpu.VMEM((TOK_PER_CHUNK, HW), jnp.int32),
        pltpu.VMEM((ROWS_PER_W, H), jnp.int32),
        pltpu.SemaphoreType.DMA,
        pltpu.SemaphoreType.DMA,
    ],
)
def _gather_mean(tok_hbm, p_hbm, out_hbm, idx_v, buf_a, buf_b, stage_v,
                 sem_a, sem_b):
    _sc_body(tok_hbm, p_hbm, out_hbm, idx_v, buf_a, buf_b, stage_v,
             sem_a, sem_b)


def _head_body(rep_ref, wsel_ref, bsel_ref, wint_ref, bint_ref,
               w1_ref, b1_ref, w2t_ref, o_ref):
    rep_b = rep_ref[...].astype(jnp.float32) * DECODE
    sel = jnp.dot(rep_b, wsel_ref[...],
                  preferred_element_type=jnp.float32) + bsel_ref[...]
    norm = jnp.sqrt(jnp.sum(sel * sel, axis=1, keepdims=True))
    seln = sel / jnp.maximum(norm, 1e-12)
    cddp = seln[0:CDD_P]
    hisp = seln[CDD_P:SLOTS]
    attn = lax.dot_general(cddp, hisp, (((1,), (1,)), ((), ())),
                           preferred_element_type=jnp.float32)

    col = lax.broadcasted_iota(jnp.int32, (CDD_P, HIS_P), 1)
    a = jnp.where(col < HIS, attn, -1e30)
    w = jnp.zeros((CDD_P, HIS_P), jnp.float32)
    for _ in range(K):
        m = jnp.max(a, axis=1, keepdims=True)
        eq = a == m
        first = jnp.min(jnp.where(eq, col, HIS_P), axis=1, keepdims=True)
        onehot = col == first
        w = w + jnp.where(onehot & (m >= THRESHOLD), m, 0.0)
        a = jnp.where(onehot, -1e30, a)

    rvalid = lax.broadcasted_iota(jnp.int32, (HIS_P, H), 0) < HIS
    his_real = jnp.where(rvalid, rep_b[CDD_P:SLOTS], 0.0)
    hisv = jnp.dot(w, his_real,
                   preferred_element_type=jnp.float32) * (1.0 / K)
    fus = jnp.maximum(
        jnp.dot(rep_b[0:CDD_P] * hisv, wint_ref[...],
                preferred_element_type=jnp.float32) + bint_ref[...], 0.0)
    h1 = jnp.maximum(
        jnp.dot(fus, w1_ref[...],
                preferred_element_type=jnp.float32) + b1_ref[...], 0.0)
    score = lax.dot_general(w2t_ref[...], h1, (((1,), (1,)), ((), ())),
                            preferred_element_type=jnp.float32)

    lane = lax.broadcasted_iota(jnp.int32, (1, CDD_P), 1)
    sm = jnp.where(lane < CDD, score, -1e30)
    mx = jnp.max(sm, axis=1, keepdims=True)
    lse = jnp.log(jnp.sum(jnp.exp(sm - mx), axis=1, keepdims=True)) + mx
    o_ref[...] = (score - lse)[None]


def _head(rep, w_sel, b_sel, w_int, b_int, w1, b1, w2):
    return pl.pallas_call(
        _head_body,
        grid=(B,),
        in_specs=[
            pl.BlockSpec((SLOTS, H), lambda b: (b, 0)),
            pl.BlockSpec((H, H), lambda b: (0, 0)),
            pl.BlockSpec((1, H), lambda b: (0, 0)),
            pl.BlockSpec((H, H), lambda b: (0, 0)),
            pl.BlockSpec((1, H), lambda b: (0, 0)),
            pl.BlockSpec((H, H // 2), lambda b: (0, 0)),
            pl.BlockSpec((1, H // 2), lambda b: (0, 0)),
            pl.BlockSpec((1, H // 2), lambda b: (0, 0)),
        ],
        out_specs=pl.BlockSpec((1, 1, CDD_P), lambda b: (b, 0, 0)),
        out_shape=jax.ShapeDtypeStruct((B, 1, CDD_P), jnp.float32),
    )(rep, w_sel, b_sel.reshape(1, H), w_int, b_int.reshape(1, H),
      w1, b1.reshape(1, H // 2), w2.reshape(1, H // 2))


def kernel(cdd_encoded_index, his_encoded_index, emb_table, W_enc, b_enc,
           W_sel, b_sel, W_int, b_int, W1, b1, W2, b2):
    p32 = _project_table(emb_table, W_enc, b_enc)
    zc = jnp.zeros((B, CDD_P - CDD, T), jnp.int32)
    zh = jnp.zeros((B, HIS_P - HIS, T), jnp.int32)
    tok = jnp.concatenate(
        [cdd_encoded_index.astype(jnp.int32), zc,
         his_encoded_index.astype(jnp.int32), zh], axis=1).reshape(-1)
    rep = _gather_mean(tok, p32)
    out = _head(rep, W_sel, b_sel, W_int, b_int, W1, b1, W2)
    return out.reshape(B, CDD_P)[:, :CDD]


# speedup vs baseline: 6.4092x; 6.1040x over previous
import functools

import jax
import jax.numpy as jnp
from jax import lax
from jax.experimental import pallas as pl
from jax.experimental.pallas import tpu as pltpu
from jax.experimental.pallas import tpu_sc as plsc

B, CDD, HIS, T, H, K = 64, 5, 50, 20, 256, 5
VOCAB = 30000
THRESHOLD = 0.1

CDD_P = 8
HIS_P = 56
SLOTS = CDD_P + HIS_P
NROWS = B * SLOTS
NW = 32
ROWS_PER_W = NROWS // NW
SEGS_PER_CHUNK = 4
TOK_PER_CHUNK = SEGS_PER_CHUNK * T
CHUNKS_PER_W = ROWS_PER_W // SEGS_PER_CHUNK
TOK_PER_W = ROWS_PER_W * T
HW = H // 2
QSCALE = float(1 << 14)
DECODE = 1.0 / float(1 << 30)


def _proj_body(x_ref, w_ref, b_ref, o_ref):
    h = jnp.dot(x_ref[...].astype(jnp.bfloat16),
                w_ref[...].astype(jnp.bfloat16),
                preferred_element_type=jnp.float32)
    v = jnp.tanh(h + b_ref[...]) * (1.0 / T)
    q = jnp.rint(v * QSCALE).astype(jnp.int32)
    lo = q[:, :HW] & jnp.int32(0xFFFF)
    hi = q[:, HW:] << 16
    o_ref[...] = lo | hi


def _project_table(emb_table, w_enc, b_enc):
    rows = 6000
    return pl.pallas_call(
        _proj_body,
        grid=(VOCAB // rows,),
        in_specs=[
            pl.BlockSpec((rows, H), lambda i: (i, 0)),
            pl.BlockSpec((H, H), lambda i: (0, 0)),
            pl.BlockSpec((1, H), lambda i: (0, 0)),
        ],
        out_specs=pl.BlockSpec((rows, HW), lambda i: (i, 0)),
        out_shape=jax.ShapeDtypeStruct((VOCAB, HW), jnp.int32),
    )(emb_table, w_enc, b_enc.reshape(1, H))


def _sc_body(tok_hbm, p_hbm, out_hbm, idx_v, bufs, stage_v, sems):
    wid = lax.axis_index("s") * 2 + lax.axis_index("c")
    pltpu.sync_copy(tok_hbm.at[pl.ds(wid * TOK_PER_W, TOK_PER_W)], idx_v)

    def fire(chunk, buf, sem):
        pltpu.async_copy(
            p_hbm.at[idx_v.at[pl.ds(chunk * TOK_PER_CHUNK, TOK_PER_CHUNK)]],
            buf, sem)

    def drain(buf, sem):
        pltpu.make_async_copy(p_hbm.at[pl.ds(0, TOK_PER_CHUNK)], buf,
                              sem).wait()

    def process(buf, chunk):
        for k in range(SEGS_PER_CHUNK):
            x0 = [buf[k * T, c * 16:(c + 1) * 16] for c in range(8)]
            lo = [x << 16 for x in x0]
            hi = [x & jnp.int32(-65536) for x in x0]
            for t in range(1, T):
                xt = [buf[k * T + t, c * 16:(c + 1) * 16] for c in range(8)]
                lo = [lo[c] + (xt[c] << 16) for c in range(8)]
                hi = [hi[c] + (xt[c] & jnp.int32(-65536)) for c in range(8)]
            slot = chunk * SEGS_PER_CHUNK + k
            for c in range(8):
                stage_v[slot, c * 16:(c + 1) * 16] = lo[c]
                stage_v[slot, HW + c * 16:HW + (c + 1) * 16] = hi[c]

    for p in range(7):
        fire(p, bufs[p], sems[p])

    def ring(i, _):
        for jj in range(8):
            j = 8 * i + jj
            drain(bufs[jj], sems[jj])
            process(bufs[jj], j)

            @pl.when(j + 7 < CHUNKS_PER_W)
            def _():
                fire(j + 7, bufs[(jj + 7) % 8], sems[(jj + 7) % 8])
        return 0

    lax.fori_loop(0, CHUNKS_PER_W // 8, ring, 0)
    pltpu.sync_copy(stage_v, out_hbm.at[pl.ds(wid * ROWS_PER_W, ROWS_PER_W)])


@functools.partial(
    pl.kernel,
    mesh=plsc.VectorSubcoreMesh(core_axis_name="c", subcore_axis_name="s"),
    out_type=jax.ShapeDtypeStruct((NROWS, H), jnp.int32),
    scratch_types=[
        pltpu.VMEM((TOK_PER_W,), jnp.int32),
    ] + [pltpu.VMEM((TOK_PER_CHUNK, HW), jnp.int32)] * 8
      + [pltpu.VMEM((ROWS_PER_W, H), jnp.int32)]
      + [pltpu.SemaphoreType.DMA] * 8,
)
def _gather_mean(tok_hbm, p_hbm, out_hbm, idx_v, b0, b1, b2, b3, b4, b5,
                 b6, b7, stage_v, s0, s1, s2, s3, s4, s5, s6, s7):
    _sc_body(tok_hbm, p_hbm, out_hbm, idx_v,
             (b0, b1, b2, b3, b4, b5, b6, b7), stage_v,
             (s0, s1, s2, s3, s4, s5, s6, s7))


NB = 16


def _head_body(rep_ref, wsel_ref, bsel_ref, wint_ref, bint_ref,
               w1_ref, b1_ref, w2t_ref, o_ref):
    rep_all = rep_ref[...].astype(jnp.float32) * DECODE
    sel = jnp.dot(rep_all, wsel_ref[...],
                  preferred_element_type=jnp.float32) + bsel_ref[...]
    norm = jnp.sqrt(jnp.sum(sel * sel, axis=1, keepdims=True))
    seln = sel / jnp.maximum(norm, 1e-12)

    cdd_all = jnp.concatenate(
        [seln[b * SLOTS:b * SLOTS + CDD_P] for b in range(NB)], axis=0)
    his_all = jnp.concatenate(
        [seln[b * SLOTS + CDD_P:(b + 1) * SLOTS] for b in range(NB)], axis=0)
    full = lax.dot_general(cdd_all, his_all, (((1,), (1,)), ((), ())),
                           preferred_element_type=jnp.float32)
    attn = jnp.concatenate(
        [full[b * CDD_P:(b + 1) * CDD_P, b * HIS_P:(b + 1) * HIS_P]
         for b in range(NB)], axis=0)

    col = lax.broadcasted_iota(jnp.int32, (NB * CDD_P, HIS_P), 1)
    a = jnp.where(col < HIS, attn, -1e30)
    w = jnp.zeros((NB * CDD_P, HIS_P), jnp.float32)
    for _ in range(K):
        m = jnp.max(a, axis=1, keepdims=True)
        eq = a == m
        first = jnp.min(jnp.where(eq, col, HIS_P), axis=1, keepdims=True)
        onehot = col == first
        w = w + jnp.where(onehot & (m >= THRESHOLD), m, 0.0)
        a = jnp.where(onehot, -1e30, a)

    rvalid = lax.broadcasted_iota(jnp.int32, (HIS_P, H), 0) < HIS
    fins = []
    for b in range(NB):
        base = b * SLOTS
        his_real = jnp.where(rvalid, rep_all[base + CDD_P:base + SLOTS], 0.0)
        hisv = jnp.dot(w[b * CDD_P:(b + 1) * CDD_P], his_real,
                       preferred_element_type=jnp.float32) * (1.0 / K)
        fins.append(rep_all[base:base + CDD_P] * hisv)

    fin = jnp.concatenate(fins, axis=0)
    fus = jnp.maximum(
        jnp.dot(fin, wint_ref[...],
                preferred_element_type=jnp.float32) + bint_ref[...], 0.0)
    h1 = jnp.maximum(
        jnp.dot(fus, w1_ref[...],
                preferred_element_type=jnp.float32) + b1_ref[...], 0.0)
    score = lax.dot_general(w2t_ref[...], h1, (((1,), (1,)), ((), ())),
                            preferred_element_type=jnp.float32)

    lane = lax.broadcasted_iota(jnp.int32, (1, CDD_P), 1)
    rows = []
    for b in range(NB):
        s = score[:, b * CDD_P:(b + 1) * CDD_P]
        sm = jnp.where(lane < CDD, s, -1e30)
        mx = jnp.max(sm, axis=1, keepdims=True)
        lse = jnp.log(jnp.sum(jnp.exp(sm - mx), axis=1, keepdims=True)) + mx
        rows.append(s - lse)
    o = jnp.concatenate(rows, axis=0)
    o_ref[...] = o[:, None, :]


def _head(rep, w_sel, b_sel, w_int, b_int, w1, b1, w2):
    return pl.pallas_call(
        _head_body,
        grid=(B // NB,),
        in_specs=[
            pl.BlockSpec((NB * SLOTS, H), lambda b: (b, 0)),
            pl.BlockSpec((H, H), lambda b: (0, 0)),
            pl.BlockSpec((1, H), lambda b: (0, 0)),
            pl.BlockSpec((H, H), lambda b: (0, 0)),
            pl.BlockSpec((1, H), lambda b: (0, 0)),
            pl.BlockSpec((H, H // 2), lambda b: (0, 0)),
            pl.BlockSpec((1, H // 2), lambda b: (0, 0)),
            pl.BlockSpec((1, H // 2), lambda b: (0, 0)),
        ],
        out_specs=pl.BlockSpec((NB, 1, CDD_P), lambda b: (b, 0, 0)),
        out_shape=jax.ShapeDtypeStruct((B, 1, CDD_P), jnp.float32),
    )(rep, w_sel, b_sel.reshape(1, H), w_int, b_int.reshape(1, H),
      w1, b1.reshape(1, H // 2), w2.reshape(1, H // 2))


def kernel(cdd_encoded_index, his_encoded_index, emb_table, W_enc, b_enc,
           W_sel, b_sel, W_int, b_int, W1, b1, W2, b2):
    p32 = _project_table(emb_table, W_enc, b_enc)
    zc = (jnp.arange(B * (CDD_P - CDD) * T, dtype=jnp.int32) % VOCAB
          ).reshape(B, CDD_P - CDD, T)
    zh = (jnp.arange(B * (HIS_P - HIS) * T, dtype=jnp.int32) % VOCAB
          ).reshape(B, HIS_P - HIS, T)
    tok = jnp.concatenate(
        [cdd_encoded_index.astype(jnp.int32), zc,
         his_encoded_index.astype(jnp.int32), zh], axis=1).reshape(-1)
    rep = _gather_mean(tok, p32)
    out = _head(rep, W_sel, b_sel, W_int, b_int, W1, b1, W2)
    return out.reshape(B, CDD_P)[:, :CDD]
